# Initial kernel scaffold; baseline (speedup 1.0000x reference)
import jax
import jax.numpy as jnp
from jax.experimental import pallas as pl


def kernel(x, edge_index, W, b):
    raise NotImplementedError("write your pallas kernel here")


# speedup vs baseline: 4.1623x; 4.1623x over previous
import functools

import jax
import jax.numpy as jnp
from jax import lax
from jax.experimental import pallas as pl
from jax.experimental.pallas import tpu as pltpu
from jax.experimental.pallas import tpu_sc as plsc

N = 10000
E = 320000
D = 128
D_OUT = 128

NC = 2
NS = 16
DH = D // NC
CHUNK = 128
ROWS_PER_TILE = 632
NP = NS * ROWS_PER_TILE
EPT = 20224
NCHUNK = EPT // CHUNK
EPAD = NS * EPT
DEGW = 16
ZROWS = ROWS_PER_TILE // 8

_sc_mesh = plsc.VectorSubcoreMesh(core_axis_name="c", subcore_axis_name="s")


@functools.partial(
    pl.kernel,
    out_type=[
        jax.ShapeDtypeStruct((NC, NP, DH), jnp.float32),
        jax.ShapeDtypeStruct((NP, DEGW), jnp.float32),
    ],
    mesh=_sc_mesh,
    scratch_types=[
        pltpu.VMEM_SHARED((NP, DH), jnp.float32),
        pltpu.VMEM_SHARED((NP, DEGW), jnp.float32),
        pltpu.VMEM((ZROWS, DH), jnp.float32),
        pltpu.VMEM((ROWS_PER_TILE, DEGW), jnp.float32),
        pltpu.VMEM((CHUNK, DEGW), jnp.float32),
        pltpu.VMEM((CHUNK,), jnp.int32),
        pltpu.VMEM((CHUNK,), jnp.int32),
        pltpu.VMEM((CHUNK, DH), jnp.float32),
        pltpu.SemaphoreType.DMA,
    ],
    compiler_params=pltpu.CompilerParams(use_tc_tiling_on_sc=False),
)
def _sc_aggregate(xh_hbm, src_hbm, dst_hbm, psum_hbm, pdeg_hbm,
                  acc_s, dacc_s, zf_v, zd_v, ones_v, sidx_v, didx_v,
                  rows_v, sem):
    cid = lax.axis_index("c")
    sid = lax.axis_index("s")

    zero16 = jnp.zeros((16,), jnp.float32)
    one16 = jnp.ones((16,), jnp.float32)

    def fill_zf(i, carry):
        for j in range(DH // 16):
            zf_v[i, pl.ds(16 * j, 16)] = zero16
        return carry

    lax.fori_loop(0, ZROWS, fill_zf, 0)

    def fill_zd(i, carry):
        zd_v[i, :] = zero16
        return carry

    lax.fori_loop(0, ROWS_PER_TILE, fill_zd, 0)

    def fill_ones(i, carry):
        ones_v[i, :] = one16
        return carry

    lax.fori_loop(0, CHUNK, fill_ones, 0)

    r0 = sid * ROWS_PER_TILE
    for j in range(8):
        pltpu.sync_copy(zf_v, acc_s.at[pl.ds(r0 + j * ZROWS, ZROWS)])

    @pl.when(cid == 0)
    def _():
        pltpu.sync_copy(zd_v, dacc_s.at[pl.ds(r0, ROWS_PER_TILE)])

    plsc.subcore_barrier()

    ebase = sid * EPT

    def chunk_body(i, carry):
        base = ebase + i * CHUNK
        pltpu.sync_copy(src_hbm.at[pl.ds(base, CHUNK)], sidx_v)
        pltpu.sync_copy(dst_hbm.at[pl.ds(base, CHUNK)], didx_v)
        pltpu.async_copy(xh_hbm.at[cid].at[sidx_v], rows_v, sem).wait()
        pltpu.sync_copy(rows_v, acc_s.at[didx_v], add=True)

        @pl.when(cid == 0)
        def _():
            pltpu.sync_copy(ones_v, dacc_s.at[didx_v], add=True)

        return carry

    lax.fori_loop(0, NCHUNK, chunk_body, 0)
    plsc.subcore_barrier()

    pltpu.sync_copy(acc_s.at[pl.ds(r0, ROWS_PER_TILE)],
                    psum_hbm.at[cid, pl.ds(r0, ROWS_PER_TILE)])

    @pl.when(cid == 0)
    def _():
        pltpu.sync_copy(dacc_s.at[pl.ds(r0, ROWS_PER_TILE)],
                        pdeg_hbm.at[pl.ds(r0, ROWS_PER_TILE)])


ROW_BLK = 400


def _tc_bundler(x_ref, p_ref, d_ref, w_ref, b_ref, o_ref):
    x = x_ref[...]
    p = jnp.concatenate([p_ref[0], p_ref[1]], axis=1)
    deg = d_ref[:, 0:1]
    c = p / jnp.maximum(deg, 1.0)
    w = w_ref[...]
    acc = (jnp.dot(x, w[:D], preferred_element_type=jnp.float32,
                   precision=lax.Precision.HIGHEST)
           + jnp.dot(c, w[D:], preferred_element_type=jnp.float32,
                     precision=lax.Precision.HIGHEST)
           + b_ref[...])
    nrm = jnp.sqrt(jnp.sum(acc * acc, axis=1, keepdims=True))
    o_ref[...] = jnp.maximum(acc / jnp.maximum(nrm, 1e-12), 0.0)


def kernel(x, edge_index, W, b):
    src = edge_index[0]
    dst = edge_index[1]
    pad = EPAD - E
    src_p = jnp.concatenate([src, jnp.zeros((pad,), jnp.int32)])
    dst_p = jnp.concatenate([dst, jnp.full((pad,), N, jnp.int32)])
    xh = x.reshape(N, NC, DH).transpose(1, 0, 2)

    psum, pdeg = _sc_aggregate(xh, src_p, dst_p)

    grid = N // ROW_BLK
    out = pl.pallas_call(
        _tc_bundler,
        grid=(grid,),
        in_specs=[
            pl.BlockSpec((ROW_BLK, D), lambda i: (i, 0)),
            pl.BlockSpec((NC, ROW_BLK, DH), lambda i: (0, i, 0)),
            pl.BlockSpec((ROW_BLK, DEGW), lambda i: (i, 0)),
            pl.BlockSpec((2 * D, D_OUT), lambda i: (0, 0)),
            pl.BlockSpec((1, D_OUT), lambda i: (0, 0)),
        ],
        out_specs=pl.BlockSpec((ROW_BLK, D_OUT), lambda i: (i, 0)),
        out_shape=jax.ShapeDtypeStruct((N, D_OUT), jnp.float32),
    )(x, psum, pdeg, W, b.reshape(1, D_OUT))
    return out


# speedup vs baseline: 7.6365x; 1.8347x over previous
import functools

import jax
import jax.numpy as jnp
from jax import lax
from jax.experimental import pallas as pl
from jax.experimental.pallas import tpu as pltpu
from jax.experimental.pallas import tpu_sc as plsc

N = 10000
E = 320000
D = 128
D_OUT = 128

NC = 2
NS = 16
DH = D // NC
CHUNK = 128
ROWS_PER_TILE = 632
NP = NS * ROWS_PER_TILE
EPT = 20224
NCHUNK = EPT // CHUNK
EPAD = NS * EPT
DEGW = 16
ZROWS = ROWS_PER_TILE // 8

_sc_mesh = plsc.VectorSubcoreMesh(core_axis_name="c", subcore_axis_name="s")


@functools.partial(
    pl.kernel,
    out_type=[
        jax.ShapeDtypeStruct((NC, NP, DH), jnp.float32),
        jax.ShapeDtypeStruct((NP, DEGW), jnp.float32),
    ],
    mesh=_sc_mesh,
    scratch_types=[
        pltpu.VMEM_SHARED((NP, DH), jnp.float32),
        pltpu.VMEM_SHARED((NP, DEGW), jnp.float32),
        pltpu.VMEM((ZROWS, DH), jnp.float32),
        pltpu.VMEM((ROWS_PER_TILE, DEGW), jnp.float32),
        pltpu.VMEM((CHUNK, DEGW), jnp.float32),
        pltpu.VMEM((NCHUNK, CHUNK), jnp.int32),
        pltpu.VMEM((NCHUNK, CHUNK), jnp.int32),
        pltpu.VMEM((CHUNK, DH), jnp.float32),
        pltpu.VMEM((CHUNK, DH), jnp.float32),
        pltpu.SemaphoreType.DMA,
        pltpu.SemaphoreType.DMA,
    ],
    compiler_params=pltpu.CompilerParams(use_tc_tiling_on_sc=False),
)
def _sc_aggregate(xh_hbm, src_hbm, dst_hbm, psum_hbm, pdeg_hbm,
                  acc_s, dacc_s, zf_v, zd_v, ones_v, sidx_v, didx_v,
                  rows0_v, rows1_v, sem0, sem1):
    cid = lax.axis_index("c")
    sid = lax.axis_index("s")

    zero16 = jnp.zeros((16,), jnp.float32)
    one16 = jnp.ones((16,), jnp.float32)

    def fill_zf(i, carry):
        for j in range(DH // 16):
            zf_v[i, pl.ds(16 * j, 16)] = zero16
        return carry

    lax.fori_loop(0, ZROWS, fill_zf, 0)

    def fill_zd(i, carry):
        zd_v[i, :] = zero16
        return carry

    lax.fori_loop(0, ROWS_PER_TILE, fill_zd, 0)

    def fill_ones(i, carry):
        ones_v[i, :] = one16
        return carry

    lax.fori_loop(0, CHUNK, fill_ones, 0)

    r0 = sid * ROWS_PER_TILE
    for j in range(8):
        pltpu.sync_copy(zf_v, acc_s.at[pl.ds(r0 + j * ZROWS, ZROWS)])

    @pl.when(cid == 0)
    def _():
        pltpu.sync_copy(zd_v, dacc_s.at[pl.ds(r0, ROWS_PER_TILE)])

    plsc.subcore_barrier()

    pltpu.sync_copy(src_hbm.at[sid], sidx_v)
    pltpu.sync_copy(dst_hbm.at[sid], didx_v)

    rows = (rows0_v, rows1_v)
    sems = (sem0, sem1)
    xsrc = xh_hbm.at[cid]

    pltpu.async_copy(xsrc.at[sidx_v.at[0]], rows0_v, sem0)

    def scatter(i, b):
        pltpu.make_async_copy(xsrc.at[sidx_v.at[i]], rows[b], sems[b]).wait()
        pltpu.sync_copy(rows[b], acc_s.at[didx_v.at[i]], add=True)

        @pl.when(cid == 0)
        def _():
            pltpu.sync_copy(ones_v, dacc_s.at[didx_v.at[i]], add=True)

    def chunk_body(i2, carry):
        i = i2 * 2
        pltpu.async_copy(xsrc.at[sidx_v.at[i + 1]], rows1_v, sem1)
        scatter(i, 0)

        @pl.when(i + 2 < NCHUNK)
        def _():
            pltpu.async_copy(xsrc.at[sidx_v.at[i + 2]], rows0_v, sem0)

        scatter(i + 1, 1)
        return carry

    lax.fori_loop(0, NCHUNK // 2, chunk_body, 0)
    plsc.subcore_barrier()

    pltpu.sync_copy(acc_s.at[pl.ds(r0, ROWS_PER_TILE)],
                    psum_hbm.at[cid, pl.ds(r0, ROWS_PER_TILE)])

    @pl.when(cid == 0)
    def _():
        pltpu.sync_copy(dacc_s.at[pl.ds(r0, ROWS_PER_TILE)],
                        pdeg_hbm.at[pl.ds(r0, ROWS_PER_TILE)])


ROW_BLK = 400


def _tc_bundler(x_ref, p_ref, d_ref, w_ref, b_ref, o_ref):
    x = x_ref[...]
    p = jnp.concatenate([p_ref[0], p_ref[1]], axis=1)
    deg = d_ref[:, 0:1]
    c = p / jnp.maximum(deg, 1.0)
    w = w_ref[...]
    acc = (jnp.dot(x, w[:D], preferred_element_type=jnp.float32,
                   precision=lax.Precision.HIGHEST)
           + jnp.dot(c, w[D:], preferred_element_type=jnp.float32,
                     precision=lax.Precision.HIGHEST)
           + b_ref[...])
    nrm = jnp.sqrt(jnp.sum(acc * acc, axis=1, keepdims=True))
    o_ref[...] = jnp.maximum(acc / jnp.maximum(nrm, 1e-12), 0.0)


def kernel(x, edge_index, W, b):
    src = edge_index[0]
    dst = edge_index[1]
    pad = EPAD - E
    src_p = jnp.concatenate([src, jnp.zeros((pad,), jnp.int32)])
    dst_p = jnp.concatenate([dst, jnp.full((pad,), N, jnp.int32)])
    src_p = src_p.reshape(NS, NCHUNK, CHUNK)
    dst_p = dst_p.reshape(NS, NCHUNK, CHUNK)
    xh = x.reshape(N, NC, DH).transpose(1, 0, 2)

    psum, pdeg = _sc_aggregate(xh, src_p, dst_p)

    grid = N // ROW_BLK
    out = pl.pallas_call(
        _tc_bundler,
        grid=(grid,),
        in_specs=[
            pl.BlockSpec((ROW_BLK, D), lambda i: (i, 0)),
            pl.BlockSpec((NC, ROW_BLK, DH), lambda i: (0, i, 0)),
            pl.BlockSpec((ROW_BLK, DEGW), lambda i: (i, 0)),
            pl.BlockSpec((2 * D, D_OUT), lambda i: (0, 0)),
            pl.BlockSpec((1, D_OUT), lambda i: (0, 0)),
        ],
        out_specs=pl.BlockSpec((ROW_BLK, D_OUT), lambda i: (i, 0)),
        out_shape=jax.ShapeDtypeStruct((N, D_OUT), jnp.float32),
    )(x, psum, pdeg, W, b.reshape(1, D_OUT))
    return out
